# Initial kernel scaffold; baseline (speedup 1.0000x reference)
import jax
import jax.numpy as jnp
from jax.experimental import pallas as pl


def kernel(x, edge_index, batch, W1, as1, ad1, b1, g1, be1, W2, as2, ad2, b2, g2, be2, W3, as3, ad3, b3, g3, be3, fc1W, fc1b, g4, be4, fc2W, fc2b, fc3W, fc3b):
    raise NotImplementedError("write your pallas kernel here")


# speedup vs baseline: 1.2464x; 1.2464x over previous
import functools
import jax
import jax.numpy as jnp
from jax.experimental import pallas as pl
from jax.experimental.pallas import tpu as pltpu

N = 10000
E = 160000
NG = 64


def _mm_kernel(x_ref, w_ref, o_ref):
    o_ref[...] = jnp.dot(x_ref[...], w_ref[...],
                         preferred_element_type=jnp.float32)


def _pallas_mm(x, w):
    n, k = x.shape
    _, m = w.shape
    blk_n = 1000
    return pl.pallas_call(
        _mm_kernel,
        grid=(n // blk_n,),
        in_specs=[
            pl.BlockSpec((blk_n, k), lambda i: (i, 0)),
            pl.BlockSpec((k, m), lambda i: (0, 0)),
        ],
        out_specs=pl.BlockSpec((blk_n, m), lambda i: (i, 0)),
        out_shape=jax.ShapeDtypeStruct((n, m), jnp.float32),
    )(x, w)


def _gat_conv(x, ei, W, asrc, adst, b, H, C):
    n = x.shape[0]
    h = _pallas_mm(x, W).reshape(n, H, C)
    s, d = ei[0], ei[1]
    al_s = jnp.sum(h * asrc[None, :, :], axis=-1)
    al_d = jnp.sum(h * adst[None, :, :], axis=-1)
    e = jax.nn.leaky_relu(al_s[s] + al_d[d], 0.2)
    emax = jax.ops.segment_max(e, d, num_segments=n)
    emax = jnp.where(jnp.isfinite(emax), emax, jnp.zeros_like(emax))
    ex = jnp.exp(e - emax[d])
    den = jax.ops.segment_sum(ex, d, num_segments=n)
    alpha = ex / (den[d] + 1e-16)
    out = jax.ops.segment_sum(h[s] * alpha[:, :, None], d, num_segments=n)
    return out.reshape(n, H * C) + b


def _bn(x, g, b):
    m = jnp.mean(x, axis=0)
    v = jnp.var(x, axis=0)
    return (x - m) / jnp.sqrt(v + 1e-5) * g + b


def kernel(x, edge_index, batch, W1, as1, ad1, b1, g1, be1, W2, as2, ad2, b2,
           g2, be2, W3, as3, ad3, b3, g3, be3, fc1W, fc1b, g4, be4, fc2W,
           fc2b, fc3W, fc3b):
    loops = jnp.arange(N, dtype=edge_index.dtype)
    ei = jnp.concatenate([edge_index, jnp.stack([loops, loops])], axis=1)
    x = jnp.pad(x, ((0, 0), (0, 11)))
    W1p = jnp.pad(W1, ((0, 11), (0, 0)))
    h = jax.nn.relu(_gat_conv(x, ei, W1p, as1, ad1, b1, 3, 256))
    h = _bn(h, g1, be1)
    h = jax.nn.relu(_gat_conv(h, ei, W2, as2, ad2, b2, 3, 128))
    h = _bn(h, g2, be2)
    h = jax.nn.relu(_gat_conv(h, ei, W3, as3, ad3, b3, 3, 128))
    h = _bn(h, g3, be3)
    p = jax.ops.segment_sum(h, batch, num_segments=NG)
    h = jax.nn.relu(p @ fc1W + fc1b)
    h = _bn(h, g4, be4)
    h = jax.nn.relu(h @ fc2W + fc2b)
    h = h @ fc3W + fc3b
    return jax.nn.relu(h).reshape(-1)


# speedup vs baseline: 1.3523x; 1.0850x over previous
import functools
import jax
import jax.numpy as jnp
from jax.experimental import pallas as pl
from jax.experimental.pallas import tpu as pltpu

N = 10000
E = 160000
NG = 64


def _mm_kernel(x_ref, w_ref, o_ref):
    o_ref[...] = jnp.dot(x_ref[...], w_ref[...],
                         preferred_element_type=jnp.float32)


def _pallas_mm(x, w):
    n, k = x.shape
    _, m = w.shape
    blk_n = 1000
    return pl.pallas_call(
        _mm_kernel,
        grid=(n // blk_n,),
        in_specs=[
            pl.BlockSpec((blk_n, k), lambda i: (i, 0)),
            pl.BlockSpec((k, m), lambda i: (0, 0)),
        ],
        out_specs=pl.BlockSpec((blk_n, m), lambda i: (i, 0)),
        out_shape=jax.ShapeDtypeStruct((n, m), jnp.float32),
    )(x, w)


def _gat_conv(x, ei, W, asrc, adst, b, H, C):
    n = x.shape[0]
    h = _pallas_mm(x, W).reshape(n, H, C)
    s, d = ei[0], ei[1]
    al_s = jnp.sum(h * asrc[None, :, :], axis=-1)
    al_d = jnp.sum(h * adst[None, :, :], axis=-1)
    e = jax.nn.leaky_relu(al_s[s] + al_d[d], 0.2)
    ex = jnp.exp(e)
    den = jax.ops.segment_sum(ex, d, num_segments=n)
    out = jax.ops.segment_sum(h[s] * ex[:, :, None], d, num_segments=n)
    out = out / (den[:, :, None] + 1e-16)
    return out.reshape(n, H * C) + b


def _bn(x, g, b):
    m = jnp.mean(x, axis=0)
    v = jnp.var(x, axis=0)
    return (x - m) / jnp.sqrt(v + 1e-5) * g + b


def kernel(x, edge_index, batch, W1, as1, ad1, b1, g1, be1, W2, as2, ad2, b2,
           g2, be2, W3, as3, ad3, b3, g3, be3, fc1W, fc1b, g4, be4, fc2W,
           fc2b, fc3W, fc3b):
    loops = jnp.arange(N, dtype=edge_index.dtype)
    ei = jnp.concatenate([edge_index, jnp.stack([loops, loops])], axis=1)
    x = jnp.pad(x, ((0, 0), (0, 11)))
    W1p = jnp.pad(W1, ((0, 11), (0, 0)))
    h = jax.nn.relu(_gat_conv(x, ei, W1p, as1, ad1, b1, 3, 256))
    h = _bn(h, g1, be1)
    h = jax.nn.relu(_gat_conv(h, ei, W2, as2, ad2, b2, 3, 128))
    h = _bn(h, g2, be2)
    h = jax.nn.relu(_gat_conv(h, ei, W3, as3, ad3, b3, 3, 128))
    h = _bn(h, g3, be3)
    p = jax.ops.segment_sum(h, batch, num_segments=NG)
    h = jax.nn.relu(p @ fc1W + fc1b)
    h = _bn(h, g4, be4)
    h = jax.nn.relu(h @ fc2W + fc2b)
    h = h @ fc3W + fc3b
    return jax.nn.relu(h).reshape(-1)
